# baseline (speedup 1.0000x reference)
import functools

import jax
import jax.numpy as jnp
from jax.experimental import pallas as pl
from jax.experimental.pallas import tpu as pltpu
from jax.experimental.pallas import tpu_sc as plsc

_B, _T, _D = 4096, 50, 128
_N = _B * _T
_W = 128

_DOSE_BOUNDS = (0.0, 0.1, 0.5, 1.0, 2.0, 5.0, 10.0, 20.0, 50.0, 100.0,
                200.0, 500.0, 1000.0)


def _composite_embedding(data_i, dose_f, total_i, unit_i,
                         data_table, dose_table, total_table, unit_table):
  mesh = plsc.VectorSubcoreMesh(core_axis_name="core",
                                subcore_axis_name="subcore")

  @functools.partial(
      pl.kernel,
      out_type=jax.ShapeDtypeStruct((_N, _D), jnp.float32),
      mesh=mesh,
      scratch_types=[
          pltpu.VMEM((_W, _D), jnp.float32),
          pltpu.VMEM((_W, _D), jnp.float32),
          pltpu.VMEM((_W, _D), jnp.float32),
          pltpu.VMEM((_W, _D), jnp.float32),
          pltpu.VMEM((1, _W), jnp.int32),
          pltpu.SemaphoreType.DMA,
      ],
  )
  def k(data_hbm, dose_hbm, total_hbm, unit_hbm,
        dtab_hbm, qtab_hbm, ttab_hbm, utab_hbm,
        out_hbm, bd, bq, bt, bu, qidx, sem):

    def body(di_v, do_v, ti_v, ui_v, out_v):
      @pl.loop(0, _W, step=16)
      def _(c):
        d = do_v[0, pl.ds(c, 16)]
        acc = jnp.zeros((16,), jnp.int32)
        for b in _DOSE_BOUNDS:
          acc = acc + jnp.where(d > b, 1, 0).astype(jnp.int32)
        qidx[0, pl.ds(c, 16)] = acc

      cp0 = pltpu.async_copy(dtab_hbm.at[di_v.at[0]], bd, sem)
      cp2 = pltpu.async_copy(ttab_hbm.at[ti_v.at[0]], bt, sem)
      cp0.wait()
      cp2.wait()

      @pl.loop(0, _W)
      def _(r):
        for c in range(0, _D, 16):
          slc = (pl.ds(r, 1), pl.ds(c, 16))
          out_v[slc] = bd[slc] + bt[slc]

    n_workers = 32
    n_per_worker = _N // _W // n_workers
    pltpu.emit_pipeline(
        body,
        grid=(n_workers, n_per_worker),
        in_specs=[
            pl.BlockSpec((1, _W), lambda w, i: (0, w * n_per_worker + i)),
            pl.BlockSpec((1, _W), lambda w, i: (0, w * n_per_worker + i)),
            pl.BlockSpec((1, _W), lambda w, i: (0, w * n_per_worker + i)),
            pl.BlockSpec((1, _W), lambda w, i: (0, w * n_per_worker + i)),
        ],
        out_specs=[pl.BlockSpec((_W, _D),
                                lambda w, i: (w * n_per_worker + i, 0))],
        core_axis_name=("core", "subcore"),
        dimension_semantics=(pltpu.PARALLEL, pltpu.ARBITRARY),
    )(data_hbm, dose_hbm, total_hbm, unit_hbm, out_hbm)

  return k(data_i, dose_f, total_i, unit_i,
           data_table, dose_table, total_table, unit_table)


def kernel(data, dose, total, unit, data_table, dose_table, total_table,
           unit_table):
  out = _composite_embedding(
      data.reshape(1, _N), dose.reshape(1, _N),
      total.reshape(1, _N), unit.reshape(1, _N),
      data_table, dose_table, total_table, unit_table)
  return out.reshape(_B, _T, _D)


# speedup vs baseline: 1.2109x; 1.2109x over previous
import functools

import jax
import jax.numpy as jnp
from jax import lax
from jax.experimental import pallas as pl
from jax.experimental.pallas import tpu as pltpu
from jax.experimental.pallas import tpu_sc as plsc

_B, _T, _D = 4096, 50, 128
_N = _B * _T
_NWORKERS = 32
_TPW = _N // _NWORKERS
_W = 128
_NWIN = _TPW // _W

_DOSE_BOUNDS = (0.0, 0.1, 0.5, 1.0, 2.0, 5.0, 10.0, 20.0, 50.0, 100.0,
                200.0, 500.0, 1000.0)


def _build_du_table(dose_table, unit_table):
  nq, nu = dose_table.shape[0], unit_table.shape[0]

  def body(q_ref, u_ref, o_ref):
    o_ref[...] = q_ref[...][:, None, :] + u_ref[...][None, :, :]

  out = pl.pallas_call(
      body,
      out_shape=jax.ShapeDtypeStruct((nq, nu, _D), jnp.float32),
  )(dose_table, unit_table)
  return out.reshape(nq * nu, _D)


def _composite_embedding(data_i, dose_f, total_i, unit_i,
                         data_table, du_table, total_table):
  mesh = plsc.VectorSubcoreMesh(core_axis_name="core",
                                subcore_axis_name="subcore")

  @functools.partial(
      pl.kernel,
      out_type=jax.ShapeDtypeStruct((_N, _D), jnp.float32),
      mesh=mesh,
      scratch_types=[
          pltpu.VMEM((_TPW,), jnp.int32),
          pltpu.VMEM((_TPW,), jnp.float32),
          pltpu.VMEM((_TPW,), jnp.int32),
          pltpu.VMEM((_TPW,), jnp.int32),
          pltpu.VMEM((_W,), jnp.int32),
          pltpu.VMEM((_W, _D), jnp.float32),
          pltpu.VMEM((_W, _D), jnp.float32),
          pltpu.VMEM((_W, _D), jnp.float32),
          pltpu.VMEM((_W, _D), jnp.float32),
          pltpu.SemaphoreType.DMA,
      ],
  )
  def k(data_hbm, dose_hbm, total_hbm, unit_hbm,
        dtab_hbm, dutab_hbm, ttab_hbm,
        out_hbm, di, dof, ti, ui, ci, bd, bdu, bt, ob, sem):
    wid = lax.axis_index("subcore") * 2 + lax.axis_index("core")
    base = wid * _TPW
    cp1 = pltpu.async_copy(data_hbm.at[pl.ds(base, _TPW)], di, sem)
    cp2 = pltpu.async_copy(dose_hbm.at[pl.ds(base, _TPW)], dof, sem)
    cp3 = pltpu.async_copy(total_hbm.at[pl.ds(base, _TPW)], ti, sem)
    cp4 = pltpu.async_copy(unit_hbm.at[pl.ds(base, _TPW)], ui, sem)
    cp1.wait()
    cp2.wait()
    cp3.wait()
    cp4.wait()

    @pl.loop(0, _NWIN)
    def _(w):
      t0 = w * _W
      g1 = pltpu.async_copy(dtab_hbm.at[di.at[pl.ds(t0, _W)]], bd, sem)
      g3 = pltpu.async_copy(ttab_hbm.at[ti.at[pl.ds(t0, _W)]], bt, sem)

      for g in range(_W // 16):
        s = t0 + g * 16
        d = dof[pl.ds(s, 16)]
        acc = jnp.zeros((16,), jnp.int32)
        for bound in _DOSE_BOUNDS:
          acc = acc + jnp.where(d > bound, 1, 0).astype(jnp.int32)
        ci[pl.ds(g * 16, 16)] = acc * 64 + ui[pl.ds(s, 16)]

      g2 = pltpu.async_copy(dutab_hbm.at[ci], bdu, sem)
      g1.wait()
      g2.wait()
      g3.wait()

      @pl.loop(0, _W)
      def _(r):
        for c in range(0, _D, 16):
          slc = (pl.ds(r, 1), pl.ds(c, 16))
          ob[slc] = bd[slc] + bdu[slc] + bt[slc]

      pltpu.sync_copy(ob, out_hbm.at[pl.ds(base + t0, _W)])

  return k(data_i, dose_f, total_i, unit_i,
           data_table, du_table, total_table)


def kernel(data, dose, total, unit, data_table, dose_table, total_table,
           unit_table):
  du_table = _build_du_table(dose_table, unit_table)
  out = _composite_embedding(
      data.reshape(_N), dose.reshape(_N), total.reshape(_N), unit.reshape(_N),
      data_table, du_table, total_table)
  return out.reshape(_B, _T, _D)


# speedup vs baseline: 1.2860x; 1.0620x over previous
import functools

import jax
import jax.numpy as jnp
from jax import lax
from jax.experimental import pallas as pl
from jax.experimental.pallas import tpu as pltpu
from jax.experimental.pallas import tpu_sc as plsc

_B, _T, _D = 4096, 50, 128
_N = _B * _T
_NWORKERS = 32
_TPW = _N // _NWORKERS
_W = 128
_NWIN = _TPW // _W

_DOSE_BOUNDS = (0.0, 0.1, 0.5, 1.0, 2.0, 5.0, 10.0, 20.0, 50.0, 100.0,
                200.0, 500.0, 1000.0)


def _build_du_table(dose_table, unit_table):
  nq, nu = dose_table.shape[0], unit_table.shape[0]

  def body(q_ref, u_ref, o_ref):
    o_ref[...] = q_ref[...][:, None, :] + u_ref[...][None, :, :]

  out = pl.pallas_call(
      body,
      out_shape=jax.ShapeDtypeStruct((nq, nu, _D), jnp.float32),
  )(dose_table, unit_table)
  return out.reshape(nq * nu, _D)


def _composite_embedding(data_i, dose_f, total_i, unit_i,
                         data_table, du_table, total_table):
  mesh = plsc.VectorSubcoreMesh(core_axis_name="core",
                                subcore_axis_name="subcore")

  @functools.partial(
      pl.kernel,
      out_type=jax.ShapeDtypeStruct((_N, _D), jnp.float32),
      mesh=mesh,
      scratch_types=[
          pltpu.VMEM((_TPW,), jnp.int32),
          pltpu.VMEM((_TPW,), jnp.float32),
          pltpu.VMEM((_TPW,), jnp.int32),
          pltpu.VMEM((_TPW,), jnp.int32),
          pltpu.VMEM((2, _W), jnp.int32),
          pltpu.VMEM((2, _W, _D), jnp.float32),
          pltpu.VMEM((2, _W, _D), jnp.float32),
          pltpu.VMEM((2, _W, _D), jnp.float32),
          pltpu.SemaphoreType.DMA((2,)),
          pltpu.SemaphoreType.DMA((2,)),
      ],
  )
  def k(data_hbm, dose_hbm, total_hbm, unit_hbm,
        dtab_hbm, dutab_hbm, ttab_hbm,
        out_hbm, di, dof, ti, ui, ci, bdu, bt, ob, sem_g, sem_o):
    wid = lax.axis_index("subcore") * 2 + lax.axis_index("core")
    base = wid * _TPW
    cp1 = pltpu.async_copy(data_hbm.at[pl.ds(base, _TPW)], di, sem_o.at[0])
    cp2 = pltpu.async_copy(dose_hbm.at[pl.ds(base, _TPW)], dof, sem_o.at[0])
    cp3 = pltpu.async_copy(total_hbm.at[pl.ds(base, _TPW)], ti, sem_o.at[0])
    cp4 = pltpu.async_copy(unit_hbm.at[pl.ds(base, _TPW)], ui, sem_o.at[0])
    cp1.wait()
    cp2.wait()
    cp3.wait()
    cp4.wait()

    def fire(w, s):
      t0 = w * _W
      pltpu.make_async_copy(
          dtab_hbm.at[di.at[pl.ds(t0, _W)]], ob.at[s], sem_g.at[s]).start()
      pltpu.make_async_copy(
          ttab_hbm.at[ti.at[pl.ds(t0, _W)]], bt.at[s], sem_g.at[s]).start()
      for g in range(_W // 16):
        src = t0 + g * 16
        d = dof[pl.ds(src, 16)]
        acc = jnp.zeros((16,), jnp.int32)
        for bound in _DOSE_BOUNDS:
          acc = acc + jnp.where(d > bound, 1, 0).astype(jnp.int32)
        ci[s, pl.ds(g * 16, 16)] = acc * 64 + ui[pl.ds(src, 16)]
      pltpu.make_async_copy(
          dutab_hbm.at[ci.at[s]], bdu.at[s], sem_g.at[s]).start()

    def wait_gathers(s):
      pltpu.make_async_copy(
          dtab_hbm.at[di.at[pl.ds(0, _W)]], ob.at[s], sem_g.at[s]).wait()
      pltpu.make_async_copy(
          ttab_hbm.at[ti.at[pl.ds(0, _W)]], bt.at[s], sem_g.at[s]).wait()
      pltpu.make_async_copy(
          dutab_hbm.at[ci.at[s]], bdu.at[s], sem_g.at[s]).wait()

    def drain_out(s):
      pltpu.make_async_copy(
          ob.at[s], out_hbm.at[pl.ds(base, _W)], sem_o.at[s]).wait()

    fire(0, 0)

    @pl.loop(0, _NWIN // 2)
    def _(p):
      for s in (0, 1):
        w = 2 * p + s
        nxt = 1 - s

        @pl.when(w + 1 < _NWIN)
        def _():
          @pl.when(w >= 1)
          def _():
            drain_out(nxt)
          fire(w + 1, nxt)

        wait_gathers(s)

        @pl.loop(0, _W)
        def _(r):
          for c in range(0, _D, 16):
            slc = (s, pl.ds(r, 1), pl.ds(c, 16))
            ob[slc] = ob[slc] + bdu[slc] + bt[slc]

        pltpu.make_async_copy(
            ob.at[s], out_hbm.at[pl.ds(base + 2 * p * _W + s * _W, _W)],
            sem_o.at[s]).start()

    drain_out(0)
    drain_out(1)

  return k(data_i, dose_f, total_i, unit_i,
           data_table, du_table, total_table)


def kernel(data, dose, total, unit, data_table, dose_table, total_table,
           unit_table):
  du_table = _build_du_table(dose_table, unit_table)
  out = _composite_embedding(
      data.reshape(_N), dose.reshape(_N), total.reshape(_N), unit.reshape(_N),
      data_table, du_table, total_table)
  return out.reshape(_B, _T, _D)
